# baseline (speedup 1.0000x reference)
import functools

import jax
import jax.numpy as jnp
from jax import lax
from jax.experimental import pallas as pl
from jax.experimental.pallas import tpu as pltpu
from jax.experimental.pallas import tpu_sc as plsc

N_NODES = 10000
N_EDGES = 320000
N_GRAPHS = 128

NC = 2
NS = 16
NW = NC * NS
EPW = N_EDGES // NW
CH = 80
NCHUNK = EPW // CH
ROWS_PER_SUB = 624
ROWS_TAIL = N_NODES - NS * ROWS_PER_SUB


def _sliced_rows_copy(src_at, dst_at, s):
    base = s * ROWS_PER_SUB
    pltpu.sync_copy(src_at(base, ROWS_PER_SUB), dst_at(base, ROWS_PER_SUB))

    @pl.when(s == NS - 1)
    def _():
        tb = NS * ROWS_PER_SUB
        pltpu.sync_copy(src_at(tb, ROWS_TAIL), dst_at(tb, ROWS_TAIL))


def _sc_mesh():
    return plsc.VectorSubcoreMesh(
        core_axis_name="c", subcore_axis_name="s", num_cores=NC, num_subcores=NS
    )


_SC_PARAMS = pltpu.CompilerParams(use_tc_tiling_on_sc=False,
                                  needs_layout_passes=False)


def _worker_ids():
    c = lax.axis_index("c")
    s = lax.axis_index("s")
    return c, s


def _sc_loopea(ea, dst, zeros32):
    kfn = functools.partial(
        pl.kernel,
        out_type=jax.ShapeDtypeStruct((NC * N_NODES, 32), jnp.float32),
        mesh=_sc_mesh(),
        scratch_types=[
            pltpu.VMEM((CH * 18 + 16,), jnp.float32),
            pltpu.VMEM((CH,), jnp.int32),
            pltpu.VMEM((CH, 32), jnp.float32),
            pltpu.VMEM_SHARED((N_NODES, 32), jnp.float32),
            pltpu.SemaphoreType.DMA,
        ],
        compiler_params=_SC_PARAMS,
    )

    @kfn
    def body(ea_hbm, dst_hbm, z_hbm, out_hbm, ea_v, didx_v, s_v, acc_sh, sem):
        c, s = _worker_ids()
        wid = s * NC + c
        ebase = wid * EPW

        _sliced_rows_copy(lambda b, n: z_hbm.at[pl.ds(b, n)],
                          lambda b, n: acc_sh.at[pl.ds(b, n)], s)
        plsc.subcore_barrier()

        lane = lax.iota(jnp.int32, 16)
        mask2 = jnp.where(lane < 2, 1.0, 0.0).astype(jnp.float32)
        onehot18 = jnp.where(lane == 2, 1.0, 0.0).astype(jnp.float32)

        def chunk_body(ci, _):
            base = ebase + ci * CH
            pltpu.sync_copy(ea_hbm.at[pl.ds(base * 18, CH * 18)], ea_v.at[pl.ds(0, CH * 18)])
            pltpu.sync_copy(dst_hbm.at[pl.ds(base, CH)], didx_v)

            def edge_body(e, _):
                c0 = ea_v[pl.ds(e * 18, 16)]
                c1 = ea_v[pl.ds(e * 18 + 16, 16)]
                s_v[e, pl.ds(0, 16)] = c0
                s_v[e, pl.ds(16, 16)] = c1 * mask2 + onehot18
                return 0

            lax.fori_loop(0, CH, edge_body, 0)
            pltpu.async_copy(s_v, acc_sh.at[didx_v], sem, add=True).wait()
            return 0

        lax.fori_loop(0, NCHUNK, chunk_body, 0)
        plsc.subcore_barrier()
        _sliced_rows_copy(lambda b, n: acc_sh.at[pl.ds(b, n)],
                          lambda b, n: out_hbm.at[pl.ds(c * N_NODES + b, n)], s)

    return body(ea.reshape(-1), dst, zeros32)


def _sc_edge_pass(albf, arbf, ewp, src, dst, attp, attn, zerosB, F, BF):
    NG = BF // 32
    o = F % 32
    den_g, den_half, den_lane = F // 32, o % 2, o // 2
    CHE = 40
    NCHE = EPW // CHE

    kfn = functools.partial(
        pl.kernel,
        out_type=jax.ShapeDtypeStruct((NC * N_NODES, BF), jnp.float32),
        mesh=_sc_mesh(),
        scratch_types=[
            pltpu.VMEM((2, CHE), jnp.int32),
            pltpu.VMEM((4, CHE), jnp.int32),
            pltpu.VMEM((2, CHE, BF), jnp.bfloat16),
            pltpu.VMEM((2, CHE, BF), jnp.bfloat16),
            pltpu.VMEM((2, CHE, BF), jnp.float32),
            pltpu.VMEM((2, CHE, BF), jnp.float32),
            pltpu.VMEM((BF,), jnp.float32),
            pltpu.VMEM((BF,), jnp.float32),
            pltpu.VMEM_SHARED((N_NODES, BF), jnp.float32),
            pltpu.SemaphoreType.DMA((2,)),
            pltpu.SemaphoreType.DMA((2,)),
            pltpu.SemaphoreType.DMA((2,)),
            pltpu.SemaphoreType.DMA((2,)),
        ],
        compiler_params=_SC_PARAMS,
    )

    @kfn
    def body(albf_hbm, arbf_hbm, ewp_hbm, src_hbm, dst_hbm, attp_hbm, attn_hbm,
             z_hbm, out_hbm, sidx_v, didx_v, a16_v, c16_v, b_v, s_v,
             attp_v, attn_v, acc_sh, isem, lsem, gsem, ssem):
        c, s = _worker_ids()
        wid = s * NC + c
        ebase = wid * EPW

        _sliced_rows_copy(lambda b, n: z_hbm.at[pl.ds(b, n)],
                          lambda b, n: acc_sh.at[pl.ds(b, n)], s)
        pltpu.sync_copy(attp_hbm, attp_v)
        pltpu.sync_copy(attn_hbm, attn_v)
        plsc.subcore_barrier()

        lane = lax.iota(jnp.int32, 16)
        onehot_d = jnp.where(lane == den_lane, 1.0, 0.0).astype(jnp.float32)
        rots = [(lane + st) & 15 for st in (1, 2, 4, 8)]
        ap = [attp_v[pl.ds(16 * k, 16)] for k in range(2 * NG)]
        an = [attn_v[pl.ds(16 * k, 16)] for k in range(2 * NG)]

        def issue_idx(ci):
            sl = ci % 2
            sl4 = ci % 4
            base = ebase + ci * CHE
            pltpu.async_copy(src_hbm.at[pl.ds(base, CHE)], sidx_v.at[sl], isem.at[sl])
            pltpu.async_copy(dst_hbm.at[pl.ds(base, CHE)], didx_v.at[sl4], isem.at[sl])

        def wait_idx(ci):
            sl = ci % 2
            sl4 = ci % 4
            base = ebase + ci * CHE
            pltpu.make_async_copy(src_hbm.at[pl.ds(base, CHE)], sidx_v.at[sl], isem.at[sl]).wait()
            pltpu.make_async_copy(dst_hbm.at[pl.ds(base, CHE)], didx_v.at[sl4], isem.at[sl]).wait()

        def issue_g(ci):
            sl = ci % 2
            base = ebase + ci * CHE
            pltpu.async_copy(ewp_hbm.at[pl.ds(base, CHE)], b_v.at[sl], lsem.at[sl])
            pltpu.async_copy(albf_hbm.at[sidx_v.at[sl]], a16_v.at[sl], gsem.at[sl])
            pltpu.async_copy(arbf_hbm.at[didx_v.at[ci % 4]], c16_v.at[sl], gsem.at[sl])

        def wait_g(ci):
            sl = ci % 2
            base = ebase + ci * CHE
            pltpu.make_async_copy(ewp_hbm.at[pl.ds(base, CHE)], b_v.at[sl], lsem.at[sl]).wait()
            pltpu.make_async_copy(albf_hbm.at[sidx_v.at[sl]], a16_v.at[sl], gsem.at[sl]).wait()
            pltpu.make_async_copy(arbf_hbm.at[didx_v.at[ci % 4]], c16_v.at[sl], gsem.at[sl]).wait()

        def issue_sc(ci):
            sl = ci % 2
            pltpu.async_copy(s_v.at[sl], acc_sh.at[didx_v.at[ci % 4]], ssem.at[sl], add=True)

        def wait_sc(ci):
            sl = ci % 2
            pltpu.make_async_copy(s_v.at[sl], acc_sh.at[didx_v.at[ci % 4]], ssem.at[sl]).wait()

        def compute(ci):
            sl = ci % 2

            def edge_work(e):
                acc = jnp.zeros((16,), jnp.float32)
                aes, aos = [], []
                for g in range(NG):
                    ab = a16_v[sl, e, pl.ds(32 * g, 32)]
                    cb = c16_v[sl, e, pl.ds(32 * g, 32)]
                    ae, ao = plsc.unpack(ab, format=plsc.PackFormat.INTERLEAVED,
                                         preferred_element_type=jnp.float32)
                    ce, co = plsc.unpack(cb, format=plsc.PackFormat.INTERLEAVED,
                                         preferred_element_type=jnp.float32)
                    ue = ae + ce + b_v[sl, e, pl.ds(32 * g, 16)]
                    uo = ao + co + b_v[sl, e, pl.ds(32 * g + 16, 16)]
                    acc = acc + ap[2 * g] * jnp.maximum(ue, 0.0) + an[2 * g] * jnp.minimum(ue, 0.0)
                    acc = acc + ap[2 * g + 1] * jnp.maximum(uo, 0.0) + an[2 * g + 1] * jnp.minimum(uo, 0.0)
                    aes.append(ae)
                    aos.append(ao)
                for r in rots:
                    acc = acc + acc.at[r].get(mode="promise_in_bounds")
                t = jnp.exp(acc)
                for g in range(NG):
                    ve = aes[g] + onehot_d if (g == den_g and den_half == 0) else aes[g]
                    vo = aos[g] + onehot_d if (g == den_g and den_half == 1) else aos[g]
                    s_v[sl, e, pl.ds(32 * g, 16)] = t * ve
                    s_v[sl, e, pl.ds(32 * g + 16, 16)] = t * vo

            def edge_body(i, _):
                edge_work(2 * i)
                edge_work(2 * i + 1)
                return 0

            lax.fori_loop(0, CHE // 2, edge_body, 0)

        issue_idx(0)
        issue_idx(1)
        wait_idx(0)
        issue_g(0)

        def loop_body(ci, _):
            @pl.when(ci + 1 < NCHE)
            def _():
                wait_idx(ci + 1)
                issue_g(ci + 1)
            wait_g(ci)

            @pl.when(ci >= 2)
            def _():
                wait_sc(ci - 2)
            compute(ci)
            issue_sc(ci)

            @pl.when(ci + 2 < NCHE)
            def _():
                issue_idx(ci + 2)
            return 0

        lax.fori_loop(0, NCHE, loop_body, 0)
        wait_sc(NCHE - 2)
        wait_sc(NCHE - 1)
        plsc.subcore_barrier()
        _sliced_rows_copy(lambda b, n: acc_sh.at[pl.ds(b, n)],
                          lambda b, n: out_hbm.at[pl.ds(c * N_NODES + b, n)], s)

    return body(albf, arbf, ewp, src, dst, attp, attn, zerosB)


def _sc_logits(albf, arbf, ewp, src, dst, attp, attn, FP):
    NG = FP // 32

    kfn = functools.partial(
        pl.kernel,
        out_type=jax.ShapeDtypeStruct((N_EDGES,), jnp.float32),
        mesh=_sc_mesh(),
        scratch_types=[
            pltpu.VMEM((2, CH), jnp.int32),
            pltpu.VMEM((2, CH), jnp.int32),
            pltpu.VMEM((2, CH, FP), jnp.bfloat16),
            pltpu.VMEM((2, CH, FP), jnp.bfloat16),
            pltpu.VMEM((2, CH, FP), jnp.float32),
            pltpu.VMEM((2, CH), jnp.float32),
            pltpu.VMEM((FP,), jnp.float32),
            pltpu.VMEM((FP,), jnp.float32),
            pltpu.SemaphoreType.DMA((2,)),
            pltpu.SemaphoreType.DMA((2,)),
            pltpu.SemaphoreType.DMA((2,)),
            pltpu.SemaphoreType.DMA((2,)),
        ],
        compiler_params=_SC_PARAMS,
    )

    @kfn
    def body(albf_hbm, arbf_hbm, ewp_hbm, src_hbm, dst_hbm, attp_hbm, attn_hbm,
             out_hbm, sidx_v, didx_v, a16_v, c16_v, b_v, t_v, attp_v, attn_v,
             isem, lsem, gsem, osem):
        c, s = _worker_ids()
        wid = s * NC + c
        ebase = wid * EPW

        pltpu.sync_copy(attp_hbm, attp_v)
        pltpu.sync_copy(attn_hbm, attn_v)

        lane = lax.iota(jnp.int32, 16)
        rots = [(lane + st) & 15 for st in (1, 2, 4, 8)]
        mask0 = lane == 0
        ap = [attp_v[pl.ds(16 * k, 16)] for k in range(2 * NG)]
        an = [attn_v[pl.ds(16 * k, 16)] for k in range(2 * NG)]

        def issue_idx(ci):
            sl = ci % 2
            base = ebase + ci * CH
            pltpu.async_copy(src_hbm.at[pl.ds(base, CH)], sidx_v.at[sl], isem.at[sl])
            pltpu.async_copy(dst_hbm.at[pl.ds(base, CH)], didx_v.at[sl], isem.at[sl])

        def wait_idx(ci):
            sl = ci % 2
            base = ebase + ci * CH
            pltpu.make_async_copy(src_hbm.at[pl.ds(base, CH)], sidx_v.at[sl], isem.at[sl]).wait()
            pltpu.make_async_copy(dst_hbm.at[pl.ds(base, CH)], didx_v.at[sl], isem.at[sl]).wait()

        def issue_g(ci):
            sl = ci % 2
            base = ebase + ci * CH
            pltpu.async_copy(ewp_hbm.at[pl.ds(base, CH)], b_v.at[sl], lsem.at[sl])
            pltpu.async_copy(albf_hbm.at[sidx_v.at[sl]], a16_v.at[sl], gsem.at[sl])
            pltpu.async_copy(arbf_hbm.at[didx_v.at[sl]], c16_v.at[sl], gsem.at[sl])

        def wait_g(ci):
            sl = ci % 2
            base = ebase + ci * CH
            pltpu.make_async_copy(ewp_hbm.at[pl.ds(base, CH)], b_v.at[sl], lsem.at[sl]).wait()
            pltpu.make_async_copy(albf_hbm.at[sidx_v.at[sl]], a16_v.at[sl], gsem.at[sl]).wait()
            pltpu.make_async_copy(arbf_hbm.at[didx_v.at[sl]], c16_v.at[sl], gsem.at[sl]).wait()

        def issue_out(ci):
            sl = ci % 2
            base = ebase + ci * CH
            pltpu.async_copy(t_v.at[sl], out_hbm.at[pl.ds(base, CH)], osem.at[sl])

        def wait_out(ci):
            sl = ci % 2
            base = ebase + ci * CH
            pltpu.make_async_copy(t_v.at[sl], out_hbm.at[pl.ds(base, CH)], osem.at[sl]).wait()

        def compute(ci):
            sl = ci % 2

            def edge_work(e):
                acc = jnp.zeros((16,), jnp.float32)
                for g in range(NG):
                    ab = a16_v[sl, e, pl.ds(32 * g, 32)]
                    cb = c16_v[sl, e, pl.ds(32 * g, 32)]
                    ae, ao = plsc.unpack(ab, format=plsc.PackFormat.INTERLEAVED,
                                         preferred_element_type=jnp.float32)
                    ce, co = plsc.unpack(cb, format=plsc.PackFormat.INTERLEAVED,
                                         preferred_element_type=jnp.float32)
                    ue = ae + ce + b_v[sl, e, pl.ds(32 * g, 16)]
                    uo = ao + co + b_v[sl, e, pl.ds(32 * g + 16, 16)]
                    acc = acc + ap[2 * g] * jnp.maximum(ue, 0.0) + an[2 * g] * jnp.minimum(ue, 0.0)
                    acc = acc + ap[2 * g + 1] * jnp.maximum(uo, 0.0) + an[2 * g + 1] * jnp.minimum(uo, 0.0)
                for r in rots:
                    acc = acc + acc.at[r].get(mode="promise_in_bounds")
                t = jnp.exp(acc)
                eidx = lane * 0 + e
                plsc.store_scatter(t_v.at[sl], [eidx], t, mask=mask0)

            def edge_body(i, _):
                edge_work(2 * i)
                edge_work(2 * i + 1)
                return 0

            lax.fori_loop(0, CH // 2, edge_body, 0)

        issue_idx(0)
        issue_idx(1)
        wait_idx(0)
        issue_g(0)

        def loop_body(ci, _):
            @pl.when(ci + 1 < NCHUNK)
            def _():
                wait_idx(ci + 1)
                issue_g(ci + 1)
            wait_g(ci)

            @pl.when(ci >= 2)
            def _():
                wait_out(ci - 2)
            compute(ci)
            issue_out(ci)

            @pl.when(ci + 2 < NCHUNK)
            def _():
                issue_idx(ci + 2)
            return 0

        lax.fori_loop(0, NCHUNK, loop_body, 0)
        wait_out(NCHUNK - 2)
        wait_out(NCHUNK - 1)

    return body(albf, arbf, ewp, src, dst, attp, attn)


def _sc_scatter_half(table, tvals, src, dst, zeros112, den_col):
    W = 112
    KC = W // 16
    GR = CH // 16

    kfn = functools.partial(
        pl.kernel,
        out_type=jax.ShapeDtypeStruct((NC * N_NODES, W), jnp.float32),
        mesh=_sc_mesh(),
        scratch_types=[
            pltpu.VMEM((2, CH), jnp.int32),
            pltpu.VMEM((4, CH), jnp.int32),
            pltpu.VMEM((2, CH), jnp.float32),
            pltpu.VMEM((2, CH, W), jnp.float32),
            pltpu.VMEM((2, CH, W), jnp.float32),
            pltpu.VMEM_SHARED((N_NODES, W), jnp.float32),
            pltpu.SemaphoreType.DMA((2,)),
            pltpu.SemaphoreType.DMA((2,)),
            pltpu.SemaphoreType.DMA((2,)),
        ],
        compiler_params=_SC_PARAMS,
    )

    @kfn
    def body(tab_hbm, t_hbm, src_hbm, dst_hbm, z_hbm, out_hbm, sidx_v, didx_v,
             t_v, a_v, s_v, acc_sh, isem, gsem, ssem):
        c, s = _worker_ids()
        wid = s * NC + c
        ebase = wid * EPW

        _sliced_rows_copy(lambda b, n: z_hbm.at[pl.ds(b, n)],
                          lambda b, n: acc_sh.at[pl.ds(b, n)], s)
        plsc.subcore_barrier()

        lane = lax.iota(jnp.int32, 16)
        if den_col is not None:
            dc_chunk, dc_lane = den_col // 16, den_col % 16
            onehot_d = jnp.where(lane == dc_lane, 1.0, 0.0).astype(jnp.float32)
        lane_consts = [lane * 0 + i for i in range(16)]

        def issue_idx(ci):
            sl = ci % 2
            sl4 = ci % 4
            base = ebase + ci * CH
            pltpu.async_copy(src_hbm.at[pl.ds(base, CH)], sidx_v.at[sl], isem.at[sl])
            pltpu.async_copy(dst_hbm.at[pl.ds(base, CH)], didx_v.at[sl4], isem.at[sl])
            pltpu.async_copy(t_hbm.at[pl.ds(base, CH)], t_v.at[sl], isem.at[sl])

        def wait_idx(ci):
            sl = ci % 2
            sl4 = ci % 4
            base = ebase + ci * CH
            pltpu.make_async_copy(src_hbm.at[pl.ds(base, CH)], sidx_v.at[sl], isem.at[sl]).wait()
            pltpu.make_async_copy(dst_hbm.at[pl.ds(base, CH)], didx_v.at[sl4], isem.at[sl]).wait()
            pltpu.make_async_copy(t_hbm.at[pl.ds(base, CH)], t_v.at[sl], isem.at[sl]).wait()

        def issue_g(ci):
            sl = ci % 2
            pltpu.async_copy(tab_hbm.at[sidx_v.at[sl]], a_v.at[sl], gsem.at[sl])

        def wait_g(ci):
            sl = ci % 2
            pltpu.make_async_copy(tab_hbm.at[sidx_v.at[sl]], a_v.at[sl], gsem.at[sl]).wait()

        def issue_sc(ci):
            sl = ci % 2
            pltpu.async_copy(s_v.at[sl], acc_sh.at[didx_v.at[ci % 4]], ssem.at[sl], add=True)

        def wait_sc(ci):
            sl = ci % 2
            pltpu.make_async_copy(s_v.at[sl], acc_sh.at[didx_v.at[ci % 4]], ssem.at[sl]).wait()

        def compute(ci):
            sl = ci % 2

            def group_body(g, _):
                tg = t_v[sl, pl.ds(g * 16, 16)]
                for e16 in range(16):
                    tv = tg.at[lane_consts[e16]].get(mode="promise_in_bounds")
                    e = g * 16 + e16
                    for k in range(KC):
                        val = a_v[sl, e, pl.ds(16 * k, 16)]
                        if den_col is not None and k == dc_chunk:
                            val = val + onehot_d
                        s_v[sl, e, pl.ds(16 * k, 16)] = tv * val
                return 0

            lax.fori_loop(0, GR, group_body, 0)

        issue_idx(0)
        issue_idx(1)
        wait_idx(0)
        issue_g(0)

        def loop_body(ci, _):
            @pl.when(ci + 1 < NCHUNK)
            def _():
                wait_idx(ci + 1)
                issue_g(ci + 1)
            wait_g(ci)

            @pl.when(ci >= 2)
            def _():
                wait_sc(ci - 2)
            compute(ci)
            issue_sc(ci)

            @pl.when(ci + 2 < NCHUNK)
            def _():
                issue_idx(ci + 2)
            return 0

        lax.fori_loop(0, NCHUNK, loop_body, 0)
        wait_sc(NCHUNK - 2)
        wait_sc(NCHUNK - 1)
        plsc.subcore_barrier()
        _sliced_rows_copy(lambda b, n: acc_sh.at[pl.ds(b, n)],
                          lambda b, n: out_hbm.at[pl.ds(c * N_NODES + b, n)], s)

    return body(table, tvals, src, dst, zeros112)


def _mm(x, w, b=None, bm=4000):
    M, K = x.shape
    Fo = w.shape[1]
    grid = (M + bm - 1) // bm

    def kern(x_ref, w_ref, b_ref, o_ref):
        acc = jnp.dot(x_ref[...], w_ref[...], preferred_element_type=jnp.float32)
        if b_ref is not None:
            acc = acc + b_ref[...]
        o_ref[...] = acc

    if b is None:
        def kern2(x_ref, w_ref, o_ref):
            kern(x_ref, w_ref, None, o_ref)
        in_specs = [
            pl.BlockSpec((bm, K), lambda i: (i, 0)),
            pl.BlockSpec((K, Fo), lambda i: (0, 0)),
        ]
        args = (x, w)
        f = kern2
    else:
        in_specs = [
            pl.BlockSpec((bm, K), lambda i: (i, 0)),
            pl.BlockSpec((K, Fo), lambda i: (0, 0)),
            pl.BlockSpec((1, Fo), lambda i: (0, 0)),
        ]
        args = (x, w, b)
        f = kern

    return pl.pallas_call(
        f,
        grid=(grid,),
        in_specs=in_specs,
        out_specs=pl.BlockSpec((bm, Fo), lambda i: (i, 0)),
        out_shape=jax.ShapeDtypeStruct((M, Fo), jnp.float32),
    )(*args)


def _tc_combine(P, ls, xlp, xrp, We32p, attrow, biasrow, F, FP, Pb=None,
                den_col=None, bn=2000):
    grid = N_NODES // bn

    def compute(nd, ndb, lsum, xl, xr, we, att, bias):
        lane32 = lax.broadcasted_iota(jnp.int32, (bn, 32), 1)
        onehot18 = jnp.where(lane32 == 18, 1.0, 0.0)
        cnt = jnp.sum(lsum * onehot18, axis=1, keepdims=True)
        ls_mean = lsum / jnp.maximum(cnt, 1.0)
        loopW = jnp.dot(ls_mean, we, preferred_element_type=jnp.float32)

        u = xl + xr + loopW
        h = jnp.where(u > 0, u, 0.2 * u)
        logit = jnp.sum(h * att, axis=1, keepdims=True)
        ex = jnp.exp(logit)

        if ndb is None:
            lane = lax.broadcasted_iota(jnp.int32, (bn, FP), 1)
            num = nd * jnp.where(lane < F, 1.0, 0.0)
            den = jnp.sum(nd * jnp.where(lane == F, 1.0, 0.0), axis=1,
                          keepdims=True)
        else:
            num = jnp.concatenate([nd, ndb[:, :FP - 112]], axis=1)
            laneb = lax.broadcasted_iota(jnp.int32, (bn, 112), 1)
            den = jnp.sum(ndb * jnp.where(laneb == den_col, 1.0, 0.0), axis=1,
                          keepdims=True)
        out = (num + ex * xl) / (den + ex + 1e-16) + bias
        return jnp.maximum(out, 0.0)

    if Pb is None:
        def kern(p0, p1, l0, l1, xl_r, xr_r, we_r, att_r, bias_r, o_ref):
            o_ref[...] = compute(p0[...] + p1[...], None, l0[...] + l1[...],
                                 xl_r[...], xr_r[...], we_r[...], att_r[...],
                                 bias_r[...])
        extra_specs = []
        extra_args = []
    else:
        def kern(p0, p1, pb0, pb1, l0, l1, xl_r, xr_r, we_r, att_r, bias_r,
                 o_ref):
            o_ref[...] = compute(p0[...] + p1[...], pb0[...] + pb1[...],
                                 l0[...] + l1[...], xl_r[...], xr_r[...],
                                 we_r[...], att_r[...], bias_r[...])
        extra_specs = [
            pl.BlockSpec((bn, 112), lambda i: (i, 0)),
            pl.BlockSpec((bn, 112), lambda i: (i + grid, 0)),
        ]
        extra_args = [Pb, Pb]

    PW = P.shape[1]
    return pl.pallas_call(
        kern,
        grid=(grid,),
        in_specs=[
            pl.BlockSpec((bn, PW), lambda i: (i, 0)),
            pl.BlockSpec((bn, PW), lambda i: (i + grid, 0)),
            *extra_specs,
            pl.BlockSpec((bn, 32), lambda i: (i, 0)),
            pl.BlockSpec((bn, 32), lambda i: (i + grid, 0)),
            pl.BlockSpec((bn, FP), lambda i: (i, 0)),
            pl.BlockSpec((bn, FP), lambda i: (i, 0)),
            pl.BlockSpec((32, FP), lambda i: (0, 0)),
            pl.BlockSpec((1, FP), lambda i: (0, 0)),
            pl.BlockSpec((1, FP), lambda i: (0, 0)),
        ],
        out_specs=pl.BlockSpec((bn, FP), lambda i: (i, 0)),
        out_shape=jax.ShapeDtypeStruct((N_NODES, FP), jnp.float32),
    )(P, P, *extra_args, ls, ls, xlp, xrp, We32p, attrow, biasrow)


def _tc_head(h2p, batch3d, W3p, b3, W4, b4, W5, b5, W6, b6, bn=2000):
    grid = N_NODES // bn
    G = N_GRAPHS

    def kern(h_ref, bat_ref, w3_ref, b3_ref, w4_ref, b4_ref, w5_ref, b5_ref,
             w6_ref, b6_ref, o_ref, acc, cnt):
        i = pl.program_id(0)

        @pl.when(i == 0)
        def _():
            acc[...] = jnp.zeros_like(acc)
            cnt[...] = jnp.zeros_like(cnt)

        z = jnp.dot(h_ref[...], w3_ref[...], preferred_element_type=jnp.float32) + b3_ref[...]
        ids = bat_ref[...].reshape(1, bn)
        gid = lax.broadcasted_iota(jnp.int32, (G, 1), 0)
        oh = jnp.where(ids == gid, 1.0, 0.0)
        acc[...] = acc[...] + jnp.dot(oh, z, preferred_element_type=jnp.float32)
        cnt[...] = cnt[...] + jnp.sum(oh, axis=1, keepdims=True)

        @pl.when(i == grid - 1)
        def _():
            p = acc[...] / jnp.maximum(cnt[...], 1.0)
            q = jnp.maximum(jnp.dot(p, w4_ref[...], preferred_element_type=jnp.float32) + b4_ref[...], 0.0)
            r = jnp.maximum(jnp.dot(q, w5_ref[...], preferred_element_type=jnp.float32) + b5_ref[...], 0.0)
            o_ref[...] = jnp.dot(r, w6_ref[...], preferred_element_type=jnp.float32) + b6_ref[...]

    FPh = h2p.shape[1]
    return pl.pallas_call(
        kern,
        grid=(grid,),
        in_specs=[
            pl.BlockSpec((bn, FPh), lambda i: (i, 0)),
            pl.BlockSpec((1, 1, bn), lambda i: (i, 0, 0)),
            pl.BlockSpec(W3p.shape, lambda i: (0, 0)),
            pl.BlockSpec((1, 400), lambda i: (0, 0)),
            pl.BlockSpec((400, 200), lambda i: (0, 0)),
            pl.BlockSpec((1, 200), lambda i: (0, 0)),
            pl.BlockSpec((200, 100), lambda i: (0, 0)),
            pl.BlockSpec((1, 100), lambda i: (0, 0)),
            pl.BlockSpec((100, 100), lambda i: (0, 0)),
            pl.BlockSpec((1, 100), lambda i: (0, 0)),
        ],
        out_specs=pl.BlockSpec((G, 100), lambda i: (0, 0)),
        out_shape=jax.ShapeDtypeStruct((G, 100), jnp.float32),
        scratch_shapes=[
            pltpu.VMEM((G, 400), jnp.float32),
            pltpu.VMEM((G, 1), jnp.float32),
        ],
    )(h2p, batch3d, W3p, b3, W4, b4, W5, b5, W6, b6)


def _pad_cols(w, fp):
    return jnp.pad(w, ((0, 0), (0, fp - w.shape[1])))


def kernel(x, edge_index, edge_attr, batch, Wl1, bl1, Wr1, br1, We1, att1,
           bias1, Wl2, bl2, Wr2, br2, We2, att2, bias2, W3, b3, W4, b4, W5,
           b5, W6, b6):
    F1, FP1 = 100, 112
    F2, FP2 = 200, 208
    src = edge_index[0]
    dst = edge_index[1]

    Wl1p = _pad_cols(Wl1, FP1)
    Wr1p = _pad_cols(Wr1, FP1)
    bl1r = _pad_cols(bl1[None, :], FP1)
    br1r = _pad_cols(br1[None, :], FP1)
    We1_32 = jnp.pad(_pad_cols(We1, FP1), ((0, 32 - 18), (0, 0)))
    att1p = jnp.pad(att1, (0, FP1 - F1))
    bias1r = _pad_cols(bias1[None, :], FP1)
    BF1 = 128
    perm1 = sum([[32 * g + 2 * i for i in range(16)]
                 + [32 * g + 2 * i + 1 for i in range(16)] for g in range(BF1 // 32)], [])
    perm1 = jnp.asarray(perm1, jnp.int32)
    We1sc = jnp.pad(We1, ((0, 0), (0, BF1 - F1)))[:, perm1]
    att1sc = jnp.pad(att1, (0, BF1 - F1))[perm1]
    att1scn = 0.2 * att1sc
    def _sc_pos(cc):
        g, oo = cc // 32, cc % 32
        return 32 * g + (oo // 2 if oo % 2 == 0 else 16 + oo // 2)
    sel1 = jnp.asarray([_sc_pos(cc) for cc in range(FP1)], jnp.int32)

    Wl2p = jnp.pad(Wl2, ((0, FP1 - F1), (0, FP2 - F2)))
    Wr2p = jnp.pad(Wr2, ((0, FP1 - F1), (0, FP2 - F2)))
    bl2r = _pad_cols(bl2[None, :], FP2)
    br2r = _pad_cols(br2[None, :], FP2)
    We2p = _pad_cols(We2, FP2)
    We2_32 = jnp.pad(We2p, ((0, 32 - 18), (0, 0)))
    att2p = jnp.pad(att2, (0, FP2 - F2))
    bias2r = _pad_cols(bias2[None, :], FP2)
    BF2 = 224
    perm2 = sum([[32 * g + 2 * i for i in range(16)]
                 + [32 * g + 2 * i + 1 for i in range(16)] for g in range(BF2 // 32)], [])
    perm2 = jnp.asarray(perm2, jnp.int32)
    We2sc = jnp.pad(We2, ((0, 0), (0, BF2 - F2)))[:, perm2]
    att2sc = jnp.pad(att2, (0, BF2 - F2))[perm2]
    att2scn = 0.2 * att2sc

    W3p = jnp.pad(W3, ((0, FP2 - F2), (0, 0)))
    b3r = b3[None, :]
    b4r = b4[None, :]
    b5r = b5[None, :]
    b6r = b6[None, :]
    batch3d = batch.reshape(N_NODES // 2000, 1, 2000)

    zeros32 = jnp.zeros((N_NODES, 32), jnp.float32)

    ewp1 = _mm(edge_attr, We1sc)
    ewp2 = _mm(edge_attr, We2sc)
    xl1p = _mm(x, Wl1p, bl1r, bm=2000)
    xr1p = _mm(x, Wr1p, br1r, bm=2000)

    ls = _sc_loopea(edge_attr, dst, zeros32)

    xl1bf = jnp.pad(xl1p, ((0, 0), (0, BF1 - FP1))).astype(jnp.bfloat16)
    xr1bf = jnp.pad(xr1p, ((0, 0), (0, BF1 - FP1))).astype(jnp.bfloat16)
    zerosB1 = jnp.zeros((N_NODES, BF1), jnp.float32)
    P1sc = _sc_edge_pass(xl1bf, xr1bf, ewp1, src, dst, att1sc, att1scn,
                         zerosB1, F1, BF1)
    P1 = P1sc[:, sel1]
    h1p = _tc_combine(P1, ls, xl1p, xr1p, We1_32, att1p[None, :], bias1r,
                      F1, FP1)

    xl2p = _mm(h1p, Wl2p, bl2r, bm=2000)
    xr2p = _mm(h1p, Wr2p, br2r, bm=2000)
    xl2bf = jnp.pad(xl2p, ((0, 0), (0, BF2 - FP2))).astype(jnp.bfloat16)
    xr2bf = jnp.pad(xr2p, ((0, 0), (0, BF2 - FP2))).astype(jnp.bfloat16)
    t2 = _sc_logits(xl2bf, xr2bf, ewp2, src, dst, att2sc, att2scn, BF2)
    xl2a = xl2p[:, :112]
    xl2b = jnp.pad(xl2p[:, 112:], ((0, 0), (0, 16)))
    zeros112 = jnp.zeros((N_NODES, 112), jnp.float32)
    P2a = _sc_scatter_half(xl2a, t2, src, dst, zeros112, None)
    P2b = _sc_scatter_half(xl2b, t2, src, dst, zeros112, 96)
    h2p = _tc_combine(P2a, ls, xl2p, xr2p, We2_32, att2p[None, :], bias2r,
                      F2, FP2, Pb=P2b, den_col=96)

    return _tc_head(h2p, batch3d, W3p, b3r, W4, b4r, W5, b5r, W6, b6r)


# speedup vs baseline: 1.1427x; 1.1427x over previous
import functools

import jax
import jax.numpy as jnp
from jax import lax
from jax.experimental import pallas as pl
from jax.experimental.pallas import tpu as pltpu
from jax.experimental.pallas import tpu_sc as plsc

N_NODES = 10000
N_EDGES = 320000
N_GRAPHS = 128

NC = 2
NS = 16
NW = NC * NS
EPW = N_EDGES // NW
CH = 80
NCHUNK = EPW // CH
ROWS_PER_SUB = 624
ROWS_TAIL = N_NODES - NS * ROWS_PER_SUB


def _sliced_rows_copy(src_at, dst_at, s):
    base = s * ROWS_PER_SUB
    pltpu.sync_copy(src_at(base, ROWS_PER_SUB), dst_at(base, ROWS_PER_SUB))

    @pl.when(s == NS - 1)
    def _():
        tb = NS * ROWS_PER_SUB
        pltpu.sync_copy(src_at(tb, ROWS_TAIL), dst_at(tb, ROWS_TAIL))


def _sc_mesh():
    return plsc.VectorSubcoreMesh(
        core_axis_name="c", subcore_axis_name="s", num_cores=NC, num_subcores=NS
    )


_SC_PARAMS = pltpu.CompilerParams(use_tc_tiling_on_sc=False,
                                  needs_layout_passes=False)


def _worker_ids():
    c = lax.axis_index("c")
    s = lax.axis_index("s")
    return c, s


def _sc_loopea(ea, dst, zeros32):
    kfn = functools.partial(
        pl.kernel,
        out_type=jax.ShapeDtypeStruct((NC * N_NODES, 32), jnp.float32),
        mesh=_sc_mesh(),
        scratch_types=[
            pltpu.VMEM((2, CH * 18 + 16), jnp.float32),
            pltpu.VMEM((4, CH), jnp.int32),
            pltpu.VMEM((2, CH, 32), jnp.float32),
            pltpu.VMEM_SHARED((N_NODES, 32), jnp.float32),
            pltpu.SemaphoreType.DMA((2,)),
            pltpu.SemaphoreType.DMA((2,)),
        ],
        compiler_params=_SC_PARAMS,
    )

    @kfn
    def body(ea_hbm, dst_hbm, z_hbm, out_hbm, ea_v, didx_v, s_v, acc_sh,
             isem, ssem):
        c, s = _worker_ids()
        wid = s * NC + c
        ebase = wid * EPW

        _sliced_rows_copy(lambda b, n: z_hbm.at[pl.ds(b, n)],
                          lambda b, n: acc_sh.at[pl.ds(b, n)], s)
        plsc.subcore_barrier()

        lane = lax.iota(jnp.int32, 16)
        mask2 = jnp.where(lane < 2, 1.0, 0.0).astype(jnp.float32)
        onehot18 = jnp.where(lane == 2, 1.0, 0.0).astype(jnp.float32)

        def issue_in(ci):
            sl = ci % 2
            base = ebase + ci * CH
            pltpu.async_copy(ea_hbm.at[pl.ds(base * 18, CH * 18)],
                             ea_v.at[sl, pl.ds(0, CH * 18)], isem.at[sl])
            pltpu.async_copy(dst_hbm.at[pl.ds(base, CH)], didx_v.at[ci % 4], isem.at[sl])

        def wait_in(ci):
            sl = ci % 2
            base = ebase + ci * CH
            pltpu.make_async_copy(ea_hbm.at[pl.ds(base * 18, CH * 18)],
                                  ea_v.at[sl, pl.ds(0, CH * 18)], isem.at[sl]).wait()
            pltpu.make_async_copy(dst_hbm.at[pl.ds(base, CH)], didx_v.at[ci % 4], isem.at[sl]).wait()

        def issue_sc(ci):
            sl = ci % 2
            pltpu.async_copy(s_v.at[sl], acc_sh.at[didx_v.at[ci % 4]], ssem.at[sl], add=True)

        def wait_sc(ci):
            sl = ci % 2
            pltpu.make_async_copy(s_v.at[sl], acc_sh.at[didx_v.at[ci % 4]], ssem.at[sl]).wait()

        def compute(ci):
            sl = ci % 2

            def edge_body(e, _):
                c0 = ea_v[sl, pl.ds(e * 18, 16)]
                c1 = ea_v[sl, pl.ds(e * 18 + 16, 16)]
                s_v[sl, e, pl.ds(0, 16)] = c0
                s_v[sl, e, pl.ds(16, 16)] = c1 * mask2 + onehot18
                return 0

            lax.fori_loop(0, CH, edge_body, 0)

        issue_in(0)
        issue_in(1)

        def loop_body(ci, _):
            wait_in(ci)

            @pl.when(ci >= 2)
            def _():
                wait_sc(ci - 2)
            compute(ci)
            issue_sc(ci)

            @pl.when(ci + 2 < NCHUNK)
            def _():
                issue_in(ci + 2)
            return 0

        lax.fori_loop(0, NCHUNK, loop_body, 0)
        wait_sc(NCHUNK - 2)
        wait_sc(NCHUNK - 1)
        plsc.subcore_barrier()
        _sliced_rows_copy(lambda b, n: acc_sh.at[pl.ds(b, n)],
                          lambda b, n: out_hbm.at[pl.ds(c * N_NODES + b, n)], s)

    return body(ea.reshape(-1), dst, zeros32)


def _sc_edge_pass(alp, arp, ewp, src, dst, attp, attn, zerosF, F, FP):
    KC = FP // 16
    t_chunk = F // 16
    t_lane = F % 16

    kfn = functools.partial(
        pl.kernel,
        out_type=jax.ShapeDtypeStruct((NC * N_NODES, FP), jnp.float32),
        mesh=_sc_mesh(),
        scratch_types=[
            pltpu.VMEM((2, CH), jnp.int32),
            pltpu.VMEM((4, CH), jnp.int32),
            pltpu.VMEM((2, CH, FP), jnp.float32),
            pltpu.VMEM((2, CH, FP), jnp.float32),
            pltpu.VMEM((2, CH, FP), jnp.float32),
            pltpu.VMEM((FP,), jnp.float32),
            pltpu.VMEM((FP,), jnp.float32),
            pltpu.VMEM_SHARED((N_NODES, FP), jnp.float32),
            pltpu.SemaphoreType.DMA((2,)),
            pltpu.SemaphoreType.DMA((2,)),
            pltpu.SemaphoreType.DMA((2,)),
            pltpu.SemaphoreType.DMA((2,)),
            pltpu.SemaphoreType.DMA((2,)),
        ],
        compiler_params=_SC_PARAMS,
    )

    @kfn
    def body(alp_hbm, arp_hbm, ewp_hbm, src_hbm, dst_hbm, attp_hbm, attn_hbm,
             z_hbm, out_hbm, sidx_v, didx_v, a_v, b_v, s_v, attp_v, attn_v,
             acc_sh, isem, lsem, asem, gsem, ssem):
        c, s = _worker_ids()
        wid = s * NC + c
        ebase = wid * EPW

        _sliced_rows_copy(lambda b, n: z_hbm.at[pl.ds(b, n)],
                          lambda b, n: acc_sh.at[pl.ds(b, n)], s)
        pltpu.sync_copy(attp_hbm, attp_v)
        pltpu.sync_copy(attn_hbm, attn_v)
        plsc.subcore_barrier()

        lane = lax.iota(jnp.int32, 16)
        onehot_t = jnp.where(lane == t_lane, 1.0, 0.0).astype(jnp.float32)
        rots = [(lane + st) & 15 for st in (1, 2, 4, 8)]
        ap = [attp_v[pl.ds(16 * k, 16)] for k in range(KC)]
        an = [attn_v[pl.ds(16 * k, 16)] for k in range(KC)]

        def issue_idx(ci):
            sl = ci % 2
            sl4 = ci % 4
            base = ebase + ci * CH
            pltpu.async_copy(src_hbm.at[pl.ds(base, CH)], sidx_v.at[sl], isem.at[sl])
            pltpu.async_copy(dst_hbm.at[pl.ds(base, CH)], didx_v.at[sl4], isem.at[sl])

        def wait_idx(ci):
            sl = ci % 2
            sl4 = ci % 4
            base = ebase + ci * CH
            pltpu.make_async_copy(src_hbm.at[pl.ds(base, CH)], sidx_v.at[sl], isem.at[sl]).wait()
            pltpu.make_async_copy(dst_hbm.at[pl.ds(base, CH)], didx_v.at[sl4], isem.at[sl]).wait()

        def issue_g1(ci):
            sl = ci % 2
            base = ebase + ci * CH
            pltpu.async_copy(ewp_hbm.at[pl.ds(base, CH)], b_v.at[sl], lsem.at[sl])
            pltpu.async_copy(alp_hbm.at[sidx_v.at[sl]], a_v.at[sl], asem.at[sl])

        def wait_g1(ci):
            sl = ci % 2
            base = ebase + ci * CH
            pltpu.make_async_copy(ewp_hbm.at[pl.ds(base, CH)], b_v.at[sl], lsem.at[sl]).wait()
            pltpu.make_async_copy(ewp_hbm.at[pl.ds(base, CH)], a_v.at[sl], asem.at[sl]).wait()

        def issue_g2(ci):
            sl = ci % 2
            pltpu.async_copy(arp_hbm.at[didx_v.at[ci % 4]], b_v.at[sl], gsem.at[sl], add=True)

        def wait_g2(ci):
            sl = ci % 2
            base = ebase + ci * CH
            pltpu.make_async_copy(ewp_hbm.at[pl.ds(base, CH)], b_v.at[sl], gsem.at[sl]).wait()

        def issue_sc(ci):
            sl = ci % 2
            pltpu.async_copy(s_v.at[sl], acc_sh.at[didx_v.at[ci % 4]], ssem.at[sl], add=True)

        def wait_sc(ci):
            sl = ci % 2
            pltpu.make_async_copy(s_v.at[sl], acc_sh.at[didx_v.at[ci % 4]], ssem.at[sl]).wait()

        def compute(ci):
            sl = ci % 2

            def edge_work(e):
                acc = jnp.zeros((16,), jnp.float32)
                avals = []
                for k in range(KC):
                    a = a_v[sl, e, pl.ds(16 * k, 16)]
                    u = a + b_v[sl, e, pl.ds(16 * k, 16)]
                    acc = acc + ap[k] * jnp.maximum(u, 0.0) + an[k] * jnp.minimum(u, 0.0)
                    avals.append(a)
                for r in rots:
                    acc = acc + acc.at[r].get(mode="promise_in_bounds")
                t = jnp.exp(acc)
                for k in range(KC):
                    val = avals[k] + onehot_t if k == t_chunk else avals[k]
                    s_v[sl, e, pl.ds(16 * k, 16)] = t * val

            def edge_body(i, _):
                edge_work(2 * i)
                edge_work(2 * i + 1)
                return 0

            lax.fori_loop(0, CH // 2, edge_body, 0)

        issue_idx(0)
        issue_idx(1)
        wait_idx(0)
        issue_g1(0)
        wait_g1(0)
        issue_g2(0)

        def loop_body(ci, _):
            @pl.when(ci + 1 < NCHUNK)
            def _():
                wait_idx(ci + 1)
                issue_g1(ci + 1)
            wait_g2(ci)

            @pl.when(ci >= 2)
            def _():
                wait_sc(ci - 2)
            compute(ci)
            issue_sc(ci)

            @pl.when(ci + 2 < NCHUNK)
            def _():
                issue_idx(ci + 2)

            @pl.when(ci + 1 < NCHUNK)
            def _():
                wait_g1(ci + 1)
                issue_g2(ci + 1)
            return 0

        lax.fori_loop(0, NCHUNK, loop_body, 0)
        wait_sc(NCHUNK - 2)
        wait_sc(NCHUNK - 1)
        plsc.subcore_barrier()
        _sliced_rows_copy(lambda b, n: acc_sh.at[pl.ds(b, n)],
                          lambda b, n: out_hbm.at[pl.ds(c * N_NODES + b, n)], s)

    return body(alp, arp, ewp, src, dst, attp, attn, zerosF)


def _sc_logits(albf, arbf, ewp, src, dst, attp, attn, FP):
    NG = FP // 32

    kfn = functools.partial(
        pl.kernel,
        out_type=jax.ShapeDtypeStruct((N_EDGES,), jnp.float32),
        mesh=_sc_mesh(),
        scratch_types=[
            pltpu.VMEM((2, CH), jnp.int32),
            pltpu.VMEM((2, CH), jnp.int32),
            pltpu.VMEM((2, CH, FP), jnp.bfloat16),
            pltpu.VMEM((2, CH, FP), jnp.bfloat16),
            pltpu.VMEM((2, CH, FP), jnp.float32),
            pltpu.VMEM((2, CH), jnp.float32),
            pltpu.VMEM((FP,), jnp.float32),
            pltpu.VMEM((FP,), jnp.float32),
            pltpu.SemaphoreType.DMA((2,)),
            pltpu.SemaphoreType.DMA((2,)),
            pltpu.SemaphoreType.DMA((2,)),
            pltpu.SemaphoreType.DMA((2,)),
        ],
        compiler_params=_SC_PARAMS,
    )

    @kfn
    def body(albf_hbm, arbf_hbm, ewp_hbm, src_hbm, dst_hbm, attp_hbm, attn_hbm,
             out_hbm, sidx_v, didx_v, a16_v, c16_v, b_v, t_v, attp_v, attn_v,
             isem, lsem, gsem, osem):
        c, s = _worker_ids()
        wid = s * NC + c
        ebase = wid * EPW

        pltpu.sync_copy(attp_hbm, attp_v)
        pltpu.sync_copy(attn_hbm, attn_v)

        lane = lax.iota(jnp.int32, 16)
        rots = [(lane + st) & 15 for st in (1, 2, 4, 8)]
        mask0 = lane == 0
        ap = [attp_v[pl.ds(16 * k, 16)] for k in range(2 * NG)]
        an = [attn_v[pl.ds(16 * k, 16)] for k in range(2 * NG)]

        def issue_idx(ci):
            sl = ci % 2
            base = ebase + ci * CH
            pltpu.async_copy(src_hbm.at[pl.ds(base, CH)], sidx_v.at[sl], isem.at[sl])
            pltpu.async_copy(dst_hbm.at[pl.ds(base, CH)], didx_v.at[sl], isem.at[sl])

        def wait_idx(ci):
            sl = ci % 2
            base = ebase + ci * CH
            pltpu.make_async_copy(src_hbm.at[pl.ds(base, CH)], sidx_v.at[sl], isem.at[sl]).wait()
            pltpu.make_async_copy(dst_hbm.at[pl.ds(base, CH)], didx_v.at[sl], isem.at[sl]).wait()

        def issue_g(ci):
            sl = ci % 2
            base = ebase + ci * CH
            pltpu.async_copy(ewp_hbm.at[pl.ds(base, CH)], b_v.at[sl], lsem.at[sl])
            pltpu.async_copy(albf_hbm.at[sidx_v.at[sl]], a16_v.at[sl], gsem.at[sl])
            pltpu.async_copy(arbf_hbm.at[didx_v.at[sl]], c16_v.at[sl], gsem.at[sl])

        def wait_g(ci):
            sl = ci % 2
            base = ebase + ci * CH
            pltpu.make_async_copy(ewp_hbm.at[pl.ds(base, CH)], b_v.at[sl], lsem.at[sl]).wait()
            pltpu.make_async_copy(albf_hbm.at[sidx_v.at[sl]], a16_v.at[sl], gsem.at[sl]).wait()
            pltpu.make_async_copy(arbf_hbm.at[didx_v.at[sl]], c16_v.at[sl], gsem.at[sl]).wait()

        def issue_out(ci):
            sl = ci % 2
            base = ebase + ci * CH
            pltpu.async_copy(t_v.at[sl], out_hbm.at[pl.ds(base, CH)], osem.at[sl])

        def wait_out(ci):
            sl = ci % 2
            base = ebase + ci * CH
            pltpu.make_async_copy(t_v.at[sl], out_hbm.at[pl.ds(base, CH)], osem.at[sl]).wait()

        def compute(ci):
            sl = ci % 2

            def edge_work(e):
                acc = jnp.zeros((16,), jnp.float32)
                for g in range(NG):
                    ab = a16_v[sl, e, pl.ds(32 * g, 32)]
                    cb = c16_v[sl, e, pl.ds(32 * g, 32)]
                    ae, ao = plsc.unpack(ab, format=plsc.PackFormat.INTERLEAVED,
                                         preferred_element_type=jnp.float32)
                    ce, co = plsc.unpack(cb, format=plsc.PackFormat.INTERLEAVED,
                                         preferred_element_type=jnp.float32)
                    ue = ae + ce + b_v[sl, e, pl.ds(32 * g, 16)]
                    uo = ao + co + b_v[sl, e, pl.ds(32 * g + 16, 16)]
                    acc = acc + ap[2 * g] * jnp.maximum(ue, 0.0) + an[2 * g] * jnp.minimum(ue, 0.0)
                    acc = acc + ap[2 * g + 1] * jnp.maximum(uo, 0.0) + an[2 * g + 1] * jnp.minimum(uo, 0.0)
                for r in rots:
                    acc = acc + acc.at[r].get(mode="promise_in_bounds")
                t = jnp.exp(acc)
                eidx = lane * 0 + e
                plsc.store_scatter(t_v.at[sl], [eidx], t, mask=mask0)

            def edge_body(i, _):
                edge_work(2 * i)
                edge_work(2 * i + 1)
                return 0

            lax.fori_loop(0, CH // 2, edge_body, 0)

        issue_idx(0)
        issue_idx(1)
        wait_idx(0)
        issue_g(0)

        def loop_body(ci, _):
            @pl.when(ci + 1 < NCHUNK)
            def _():
                wait_idx(ci + 1)
                issue_g(ci + 1)
            wait_g(ci)

            @pl.when(ci >= 2)
            def _():
                wait_out(ci - 2)
            compute(ci)
            issue_out(ci)

            @pl.when(ci + 2 < NCHUNK)
            def _():
                issue_idx(ci + 2)
            return 0

        lax.fori_loop(0, NCHUNK, loop_body, 0)
        wait_out(NCHUNK - 2)
        wait_out(NCHUNK - 1)

    return body(albf, arbf, ewp, src, dst, attp, attn)


def _sc_scatter_half(table, tvals, src, dst, zeros112, den_col):
    W = 112
    KC = W // 16
    GR = CH // 16

    kfn = functools.partial(
        pl.kernel,
        out_type=jax.ShapeDtypeStruct((NC * N_NODES, W), jnp.float32),
        mesh=_sc_mesh(),
        scratch_types=[
            pltpu.VMEM((2, CH), jnp.int32),
            pltpu.VMEM((4, CH), jnp.int32),
            pltpu.VMEM((2, CH), jnp.float32),
            pltpu.VMEM((2, CH, W), jnp.float32),
            pltpu.VMEM((2, CH, W), jnp.float32),
            pltpu.VMEM_SHARED((N_NODES, W), jnp.float32),
            pltpu.SemaphoreType.DMA((2,)),
            pltpu.SemaphoreType.DMA((2,)),
            pltpu.SemaphoreType.DMA((2,)),
        ],
        compiler_params=_SC_PARAMS,
    )

    @kfn
    def body(tab_hbm, t_hbm, src_hbm, dst_hbm, z_hbm, out_hbm, sidx_v, didx_v,
             t_v, a_v, s_v, acc_sh, isem, gsem, ssem):
        c, s = _worker_ids()
        wid = s * NC + c
        ebase = wid * EPW

        _sliced_rows_copy(lambda b, n: z_hbm.at[pl.ds(b, n)],
                          lambda b, n: acc_sh.at[pl.ds(b, n)], s)
        plsc.subcore_barrier()

        lane = lax.iota(jnp.int32, 16)
        if den_col is not None:
            dc_chunk, dc_lane = den_col // 16, den_col % 16
            onehot_d = jnp.where(lane == dc_lane, 1.0, 0.0).astype(jnp.float32)
        lane_consts = [lane * 0 + i for i in range(16)]

        def issue_idx(ci):
            sl = ci % 2
            sl4 = ci % 4
            base = ebase + ci * CH
            pltpu.async_copy(src_hbm.at[pl.ds(base, CH)], sidx_v.at[sl], isem.at[sl])
            pltpu.async_copy(dst_hbm.at[pl.ds(base, CH)], didx_v.at[sl4], isem.at[sl])
            pltpu.async_copy(t_hbm.at[pl.ds(base, CH)], t_v.at[sl], isem.at[sl])

        def wait_idx(ci):
            sl = ci % 2
            sl4 = ci % 4
            base = ebase + ci * CH
            pltpu.make_async_copy(src_hbm.at[pl.ds(base, CH)], sidx_v.at[sl], isem.at[sl]).wait()
            pltpu.make_async_copy(dst_hbm.at[pl.ds(base, CH)], didx_v.at[sl4], isem.at[sl]).wait()
            pltpu.make_async_copy(t_hbm.at[pl.ds(base, CH)], t_v.at[sl], isem.at[sl]).wait()

        def issue_g(ci):
            sl = ci % 2
            pltpu.async_copy(tab_hbm.at[sidx_v.at[sl]], a_v.at[sl], gsem.at[sl])

        def wait_g(ci):
            sl = ci % 2
            pltpu.make_async_copy(tab_hbm.at[sidx_v.at[sl]], a_v.at[sl], gsem.at[sl]).wait()

        def issue_sc(ci):
            sl = ci % 2
            pltpu.async_copy(s_v.at[sl], acc_sh.at[didx_v.at[ci % 4]], ssem.at[sl], add=True)

        def wait_sc(ci):
            sl = ci % 2
            pltpu.make_async_copy(s_v.at[sl], acc_sh.at[didx_v.at[ci % 4]], ssem.at[sl]).wait()

        def compute(ci):
            sl = ci % 2

            def group_body(g, _):
                tg = t_v[sl, pl.ds(g * 16, 16)]
                for e16 in range(16):
                    tv = tg.at[lane_consts[e16]].get(mode="promise_in_bounds")
                    e = g * 16 + e16
                    for k in range(KC):
                        val = a_v[sl, e, pl.ds(16 * k, 16)]
                        if den_col is not None and k == dc_chunk:
                            val = val + onehot_d
                        s_v[sl, e, pl.ds(16 * k, 16)] = tv * val
                return 0

            lax.fori_loop(0, GR, group_body, 0)

        issue_idx(0)
        issue_idx(1)
        wait_idx(0)
        issue_g(0)

        def loop_body(ci, _):
            @pl.when(ci + 1 < NCHUNK)
            def _():
                wait_idx(ci + 1)
                issue_g(ci + 1)
            wait_g(ci)

            @pl.when(ci >= 2)
            def _():
                wait_sc(ci - 2)
            compute(ci)
            issue_sc(ci)

            @pl.when(ci + 2 < NCHUNK)
            def _():
                issue_idx(ci + 2)
            return 0

        lax.fori_loop(0, NCHUNK, loop_body, 0)
        wait_sc(NCHUNK - 2)
        wait_sc(NCHUNK - 1)
        plsc.subcore_barrier()
        _sliced_rows_copy(lambda b, n: acc_sh.at[pl.ds(b, n)],
                          lambda b, n: out_hbm.at[pl.ds(c * N_NODES + b, n)], s)

    return body(table, tvals, src, dst, zeros112)


def _mm(x, w, b=None, bm=4000):
    M, K = x.shape
    Fo = w.shape[1]
    grid = (M + bm - 1) // bm

    def kern(x_ref, w_ref, b_ref, o_ref):
        acc = jnp.dot(x_ref[...], w_ref[...], preferred_element_type=jnp.float32)
        if b_ref is not None:
            acc = acc + b_ref[...]
        o_ref[...] = acc

    if b is None:
        def kern2(x_ref, w_ref, o_ref):
            kern(x_ref, w_ref, None, o_ref)
        in_specs = [
            pl.BlockSpec((bm, K), lambda i: (i, 0)),
            pl.BlockSpec((K, Fo), lambda i: (0, 0)),
        ]
        args = (x, w)
        f = kern2
    else:
        in_specs = [
            pl.BlockSpec((bm, K), lambda i: (i, 0)),
            pl.BlockSpec((K, Fo), lambda i: (0, 0)),
            pl.BlockSpec((1, Fo), lambda i: (0, 0)),
        ]
        args = (x, w, b)
        f = kern

    return pl.pallas_call(
        f,
        grid=(grid,),
        in_specs=in_specs,
        out_specs=pl.BlockSpec((bm, Fo), lambda i: (i, 0)),
        out_shape=jax.ShapeDtypeStruct((M, Fo), jnp.float32),
    )(*args)


def _tc_combine(P, ls, xlp, xrp, We32p, attrow, biasrow, F, FP, Pb=None,
                den_col=None, bn=2000):
    grid = N_NODES // bn

    def compute(nd, ndb, lsum, xl, xr, we, att, bias):
        lane32 = lax.broadcasted_iota(jnp.int32, (bn, 32), 1)
        onehot18 = jnp.where(lane32 == 18, 1.0, 0.0)
        cnt = jnp.sum(lsum * onehot18, axis=1, keepdims=True)
        ls_mean = lsum / jnp.maximum(cnt, 1.0)
        loopW = jnp.dot(ls_mean, we, preferred_element_type=jnp.float32)

        u = xl + xr + loopW
        h = jnp.where(u > 0, u, 0.2 * u)
        logit = jnp.sum(h * att, axis=1, keepdims=True)
        ex = jnp.exp(logit)

        if ndb is None:
            lane = lax.broadcasted_iota(jnp.int32, (bn, FP), 1)
            num = nd * jnp.where(lane < F, 1.0, 0.0)
            den = jnp.sum(nd * jnp.where(lane == F, 1.0, 0.0), axis=1,
                          keepdims=True)
        else:
            num = jnp.concatenate([nd, ndb[:, :FP - 112]], axis=1)
            laneb = lax.broadcasted_iota(jnp.int32, (bn, 112), 1)
            den = jnp.sum(ndb * jnp.where(laneb == den_col, 1.0, 0.0), axis=1,
                          keepdims=True)
        out = (num + ex * xl) / (den + ex + 1e-16) + bias
        return jnp.maximum(out, 0.0)

    if Pb is None:
        def kern(p0, p1, l0, l1, xl_r, xr_r, we_r, att_r, bias_r, o_ref):
            o_ref[...] = compute(p0[...] + p1[...], None, l0[...] + l1[...],
                                 xl_r[...], xr_r[...], we_r[...], att_r[...],
                                 bias_r[...])
        extra_specs = []
        extra_args = []
    else:
        def kern(p0, p1, pb0, pb1, l0, l1, xl_r, xr_r, we_r, att_r, bias_r,
                 o_ref):
            o_ref[...] = compute(p0[...] + p1[...], pb0[...] + pb1[...],
                                 l0[...] + l1[...], xl_r[...], xr_r[...],
                                 we_r[...], att_r[...], bias_r[...])
        extra_specs = [
            pl.BlockSpec((bn, 112), lambda i: (i, 0)),
            pl.BlockSpec((bn, 112), lambda i: (i + grid, 0)),
        ]
        extra_args = [Pb, Pb]

    PW = P.shape[1]
    return pl.pallas_call(
        kern,
        grid=(grid,),
        in_specs=[
            pl.BlockSpec((bn, PW), lambda i: (i, 0)),
            pl.BlockSpec((bn, PW), lambda i: (i + grid, 0)),
            *extra_specs,
            pl.BlockSpec((bn, 32), lambda i: (i, 0)),
            pl.BlockSpec((bn, 32), lambda i: (i + grid, 0)),
            pl.BlockSpec((bn, FP), lambda i: (i, 0)),
            pl.BlockSpec((bn, FP), lambda i: (i, 0)),
            pl.BlockSpec((32, FP), lambda i: (0, 0)),
            pl.BlockSpec((1, FP), lambda i: (0, 0)),
            pl.BlockSpec((1, FP), lambda i: (0, 0)),
        ],
        out_specs=pl.BlockSpec((bn, FP), lambda i: (i, 0)),
        out_shape=jax.ShapeDtypeStruct((N_NODES, FP), jnp.float32),
    )(P, P, *extra_args, ls, ls, xlp, xrp, We32p, attrow, biasrow)


def _tc_head(h2p, batch3d, W3p, b3, W4, b4, W5, b5, W6, b6, bn=2000):
    grid = N_NODES // bn
    G = N_GRAPHS

    def kern(h_ref, bat_ref, w3_ref, b3_ref, w4_ref, b4_ref, w5_ref, b5_ref,
             w6_ref, b6_ref, o_ref, acc, cnt):
        i = pl.program_id(0)

        @pl.when(i == 0)
        def _():
            acc[...] = jnp.zeros_like(acc)
            cnt[...] = jnp.zeros_like(cnt)

        z = jnp.dot(h_ref[...], w3_ref[...], preferred_element_type=jnp.float32) + b3_ref[...]
        ids = bat_ref[...].reshape(1, bn)
        gid = lax.broadcasted_iota(jnp.int32, (G, 1), 0)
        oh = jnp.where(ids == gid, 1.0, 0.0)
        acc[...] = acc[...] + jnp.dot(oh, z, preferred_element_type=jnp.float32)
        cnt[...] = cnt[...] + jnp.sum(oh, axis=1, keepdims=True)

        @pl.when(i == grid - 1)
        def _():
            p = acc[...] / jnp.maximum(cnt[...], 1.0)
            q = jnp.maximum(jnp.dot(p, w4_ref[...], preferred_element_type=jnp.float32) + b4_ref[...], 0.0)
            r = jnp.maximum(jnp.dot(q, w5_ref[...], preferred_element_type=jnp.float32) + b5_ref[...], 0.0)
            o_ref[...] = jnp.dot(r, w6_ref[...], preferred_element_type=jnp.float32) + b6_ref[...]

    FPh = h2p.shape[1]
    return pl.pallas_call(
        kern,
        grid=(grid,),
        in_specs=[
            pl.BlockSpec((bn, FPh), lambda i: (i, 0)),
            pl.BlockSpec((1, 1, bn), lambda i: (i, 0, 0)),
            pl.BlockSpec(W3p.shape, lambda i: (0, 0)),
            pl.BlockSpec((1, 400), lambda i: (0, 0)),
            pl.BlockSpec((400, 200), lambda i: (0, 0)),
            pl.BlockSpec((1, 200), lambda i: (0, 0)),
            pl.BlockSpec((200, 100), lambda i: (0, 0)),
            pl.BlockSpec((1, 100), lambda i: (0, 0)),
            pl.BlockSpec((100, 100), lambda i: (0, 0)),
            pl.BlockSpec((1, 100), lambda i: (0, 0)),
        ],
        out_specs=pl.BlockSpec((G, 100), lambda i: (0, 0)),
        out_shape=jax.ShapeDtypeStruct((G, 100), jnp.float32),
        scratch_shapes=[
            pltpu.VMEM((G, 400), jnp.float32),
            pltpu.VMEM((G, 1), jnp.float32),
        ],
    )(h2p, batch3d, W3p, b3, W4, b4, W5, b5, W6, b6)


def _pad_cols(w, fp):
    return jnp.pad(w, ((0, 0), (0, fp - w.shape[1])))


def kernel(x, edge_index, edge_attr, batch, Wl1, bl1, Wr1, br1, We1, att1,
           bias1, Wl2, bl2, Wr2, br2, We2, att2, bias2, W3, b3, W4, b4, W5,
           b5, W6, b6):
    F1, FP1 = 100, 112
    F2, FP2 = 200, 208
    src = edge_index[0]
    dst = edge_index[1]

    Wl1p = _pad_cols(Wl1, FP1)
    Wr1p = _pad_cols(Wr1, FP1)
    bl1r = _pad_cols(bl1[None, :], FP1)
    br1r = _pad_cols(br1[None, :], FP1)
    We1p = _pad_cols(We1, FP1)
    We1_32 = jnp.pad(We1p, ((0, 32 - 18), (0, 0)))
    att1p = jnp.pad(att1, (0, FP1 - F1))
    att1n = 0.2 * att1p
    bias1r = _pad_cols(bias1[None, :], FP1)

    Wl2p = jnp.pad(Wl2, ((0, FP1 - F1), (0, FP2 - F2)))
    Wr2p = jnp.pad(Wr2, ((0, FP1 - F1), (0, FP2 - F2)))
    bl2r = _pad_cols(bl2[None, :], FP2)
    br2r = _pad_cols(br2[None, :], FP2)
    We2p = _pad_cols(We2, FP2)
    We2_32 = jnp.pad(We2p, ((0, 32 - 18), (0, 0)))
    att2p = jnp.pad(att2, (0, FP2 - F2))
    bias2r = _pad_cols(bias2[None, :], FP2)
    BF2 = 224
    perm2 = sum([[32 * g + 2 * i for i in range(16)]
                 + [32 * g + 2 * i + 1 for i in range(16)] for g in range(BF2 // 32)], [])
    perm2 = jnp.asarray(perm2, jnp.int32)
    We2sc = jnp.pad(We2, ((0, 0), (0, BF2 - F2)))[:, perm2]
    att2sc = jnp.pad(att2, (0, BF2 - F2))[perm2]
    att2scn = 0.2 * att2sc

    W3p = jnp.pad(W3, ((0, FP2 - F2), (0, 0)))
    b3r = b3[None, :]
    b4r = b4[None, :]
    b5r = b5[None, :]
    b6r = b6[None, :]
    batch3d = batch.reshape(N_NODES // 2000, 1, 2000)

    zeros32 = jnp.zeros((N_NODES, 32), jnp.float32)

    ewp1 = _mm(edge_attr, We1p)
    ewp2 = _mm(edge_attr, We2sc)
    xl1p = _mm(x, Wl1p, bl1r, bm=2000)
    xr1p = _mm(x, Wr1p, br1r, bm=2000)

    ls = _sc_loopea(edge_attr, dst, zeros32)

    zerosF1 = jnp.zeros((N_NODES, FP1), jnp.float32)
    P1 = _sc_edge_pass(xl1p, xr1p, ewp1, src, dst, att1p, att1n, zerosF1,
                       F1, FP1)
    h1p = _tc_combine(P1, ls, xl1p, xr1p, We1_32, att1p[None, :], bias1r,
                      F1, FP1)

    xl2p = _mm(h1p, Wl2p, bl2r, bm=2000)
    xr2p = _mm(h1p, Wr2p, br2r, bm=2000)
    xl2bf = jnp.pad(xl2p, ((0, 0), (0, BF2 - FP2))).astype(jnp.bfloat16)
    xr2bf = jnp.pad(xr2p, ((0, 0), (0, BF2 - FP2))).astype(jnp.bfloat16)
    t2 = _sc_logits(xl2bf, xr2bf, ewp2, src, dst, att2sc, att2scn, BF2)
    xl2a = xl2p[:, :112]
    xl2b = jnp.pad(xl2p[:, 112:], ((0, 0), (0, 16)))
    zeros112 = jnp.zeros((N_NODES, 112), jnp.float32)
    P2a = _sc_scatter_half(xl2a, t2, src, dst, zeros112, None)
    P2b = _sc_scatter_half(xl2b, t2, src, dst, zeros112, 96)
    h2p = _tc_combine(P2a, ls, xl2p, xr2p, We2_32, att2p[None, :], bias2r,
                      F2, FP2, Pb=P2b, den_col=96)

    return _tc_head(h2p, batch3d, W3p, b3r, W4, b4r, W5, b5r, W6, b6r)
